# baseline (speedup 1.0000x reference)
import functools

import jax
import jax.numpy as jnp
from jax import lax
from jax.experimental import pallas as pl
from jax.experimental.pallas import tpu as pltpu
from jax.experimental.pallas import tpu_sc as plsc

NC = 2
NS = 16
NW = NC * NS
LANES = 16
IDX_CHUNK = 128


def _skipgram_body(b_per_w, idx_in_hbm, idx_out_hbm, emb_in_hbm, emb_w_hbm,
                   bias_hbm, out_hbm, idxin_v, idxout_v, vin_v, w_v, bias_v,
                   out_v, t0_v, t1_v, sem):
    wid = lax.axis_index("s") * NC + lax.axis_index("c")
    n_in_chunks = b_per_w // IDX_CHUNK
    n_out_chunks = 2 * b_per_w // IDX_CHUNK

    pltpu.sync_copy(idx_in_hbm.at[wid], idxin_v)
    pltpu.sync_copy(idx_out_hbm.at[wid], idxout_v)

    copies = []
    for j in range(n_in_chunks):
        copies.append(pltpu.async_copy(
            emb_in_hbm.at[idxin_v.at[j]],
            vin_v.at[pl.ds(j * IDX_CHUNK, IDX_CHUNK)], sem))
    for j in range(n_out_chunks):
        copies.append(pltpu.async_copy(
            emb_w_hbm.at[idxout_v.at[j]],
            w_v.at[pl.ds(j * IDX_CHUNK, IDX_CHUNK)], sem))
    for j in range(n_out_chunks):
        copies.append(pltpu.async_copy(
            bias_hbm.at[idxout_v.at[j]],
            bias_v.at[pl.ds(j * IDX_CHUNK, IDX_CHUNK)], sem))
    for c in copies:
        c.wait()

    iota = lax.iota(jnp.int32, LANES)

    def group(g, carry):
        base = g * LANES
        for r in range(LANES):
            b = base + r
            p0 = None
            p1 = None
            for c in range(4):
                sl = pl.ds(c * LANES, LANES)
                vin_c = vin_v[b, sl]
                m0 = vin_c * w_v[2 * b, sl]
                m1 = vin_c * w_v[2 * b + 1, sl]
                p0 = m0 if p0 is None else p0 + m0
                p1 = m1 if p1 is None else p1 + m1
            t0_v[pl.ds(r * LANES, LANES)] = p0
            t1_v[pl.ds(r * LANES, LANES)] = p1
        row_base = iota * LANES
        dot0 = None
        dot1 = None
        for c in range(LANES):
            g0 = plsc.load_gather(t0_v, [row_base + c])
            g1 = plsc.load_gather(t1_v, [row_base + c])
            dot0 = g0 if dot0 is None else dot0 + g0
            dot1 = g1 if dot1 is None else dot1 + g1
        pos0 = 2 * (base + iota)
        pos1 = pos0 + 1
        t0 = dot0 + plsc.load_gather(bias_v, [pos0])
        t1 = dot1 + plsc.load_gather(bias_v, [pos1])
        s0 = 1.0 / (1.0 + jnp.exp(-t0))
        s1 = 1.0 / (1.0 + jnp.exp(-t1))
        plsc.store_scatter(out_v, [pos0], s0)
        plsc.store_scatter(out_v, [pos1], s1)
        return carry

    lax.fori_loop(0, b_per_w // LANES, group, 0)

    pltpu.sync_copy(out_v, out_hbm.at[pl.ds(wid * 2 * b_per_w, 2 * b_per_w)])


def kernel(x, emb_in, emb_out_w, emb_out_b):
    batch = x.shape[0]
    vocab, embed = emb_in.shape
    assert batch % (NW * LANES) == 0 and embed == 4 * LANES
    b_per_w = batch // NW

    idx_in = x[:, 0].reshape(NW, b_per_w // IDX_CHUNK, IDX_CHUNK)
    idx_out = x[:, 1:3].reshape(NW, 2 * b_per_w // IDX_CHUNK, IDX_CHUNK)
    bias_flat = emb_out_b.reshape(vocab)

    mesh = plsc.VectorSubcoreMesh(core_axis_name="c", subcore_axis_name="s")
    run = pl.kernel(
        functools.partial(_skipgram_body, b_per_w),
        out_type=jax.ShapeDtypeStruct((batch * 2,), jnp.float32),
        mesh=mesh,
        compiler_params=pltpu.CompilerParams(
            needs_layout_passes=False, use_tc_tiling_on_sc=False),
        scratch_types=[
            pltpu.VMEM((b_per_w // IDX_CHUNK, IDX_CHUNK), jnp.int32),
            pltpu.VMEM((2 * b_per_w // IDX_CHUNK, IDX_CHUNK), jnp.int32),
            pltpu.VMEM((b_per_w, embed), jnp.float32),
            pltpu.VMEM((2 * b_per_w, embed), jnp.float32),
            pltpu.VMEM((2 * b_per_w,), jnp.float32),
            pltpu.VMEM((2 * b_per_w,), jnp.float32),
            pltpu.VMEM((LANES * LANES,), jnp.float32),
            pltpu.VMEM((LANES * LANES,), jnp.float32),
            pltpu.SemaphoreType.DMA,
        ],
    )
    out_flat = run(idx_in, idx_out, emb_in, emb_out_w, bias_flat)
    return out_flat.reshape(batch, 2)
